# baseline (speedup 1.0000x reference)
import functools

import jax
import jax.numpy as jnp
from jax import lax
from jax.experimental import pallas as pl
from jax.experimental.pallas import tpu as pltpu
from jax.experimental.pallas import tpu_sc as plsc

_VOCAB = 100000
_DIM = 1024
_B = 16
_S = 2048
_NC = 2
_NS = 16
_NW = _NC * _NS
_S_PER_W = _S // _NW
_CHUNK = 32
_LANES = 16


def _emb_body(we_hbm, ids_hbm, pe_hbm, out_hbm, idx_v, pe_v, rows_v, sem):
    cid = lax.axis_index("c")
    sid = lax.axis_index("s")
    wid = sid * _NC + cid
    s_base = wid * _S_PER_W

    for h in range(_S_PER_W // _CHUNK):
        s0 = s_base + h * _CHUNK
        pltpu.sync_copy(pe_hbm.at[pl.ds(s0, _CHUNK)], pe_v)

        def batch_body(b, carry):
            tok = b * _S + s0
            pltpu.sync_copy(ids_hbm.at[pl.ds(tok, _CHUNK)], idx_v)
            pltpu.async_copy(we_hbm.at[idx_v], rows_v, sem).wait()

            def row_body(j, c1):
                def col_body(k, c2):
                    sl = pl.ds(k * _LANES, _LANES)
                    rows_v[j, sl] = rows_v[j, sl] + pe_v[j, sl]
                    return c2
                return lax.fori_loop(0, _DIM // _LANES, col_body, c1)

            lax.fori_loop(0, _CHUNK, row_body, 0)
            pltpu.sync_copy(rows_v, out_hbm.at[pl.ds(tok, _CHUNK)])
            return carry

        lax.fori_loop(0, _B, batch_body, 0)


@functools.partial(jax.jit, donate_argnums=())
def _emb_call(word_embeddings, ids_flat, position_embeddings):
    mesh = plsc.VectorSubcoreMesh(core_axis_name="c", subcore_axis_name="s")
    return pl.kernel(
        _emb_body,
        mesh=mesh,
        out_type=jax.ShapeDtypeStruct((_B * _S, _DIM), jnp.float32),
        scratch_types=[
            pltpu.VMEM((_CHUNK,), jnp.int32),
            pltpu.VMEM((_CHUNK, _DIM), jnp.float32),
            pltpu.VMEM((_CHUNK, _DIM), jnp.float32),
            pltpu.SemaphoreType.DMA,
        ],
    )(word_embeddings, ids_flat, position_embeddings)


def kernel(input_ids, word_embeddings, position_embeddings):
    ids_flat = jnp.asarray(input_ids, jnp.int32).reshape(-1)
    out = _emb_call(word_embeddings, ids_flat, position_embeddings)
    return out.reshape(_B, _S, _DIM)


# speedup vs baseline: 3.5589x; 3.5589x over previous
import functools

import jax
import jax.numpy as jnp
from jax import lax
from jax.experimental import pallas as pl
from jax.experimental.pallas import tpu as pltpu
from jax.experimental.pallas import tpu_sc as plsc

_VOCAB = 100000
_DIM = 1024
_B = 16
_S = 2048
_NC = 2
_NS = 16
_NW = _NC * _NS
_S_PER_W = _S // _NW
_CHUNK = 16
_NBUF = 3
_NCHUNK = _B * (_S_PER_W // _CHUNK)
_LANES = 16
_VEC_PER_CHUNK = _CHUNK * _DIM // _LANES


def _emb_body(we_hbm, ids_hbm, pe_hbm, out_hbm, idx_all, pe_v,
              r0, r1, r2, sg0, sg1, sg2, ss0, ss1, ss2, spe, sidx):
    cid = lax.axis_index("c")
    sid = lax.axis_index("s")
    wid = sid * _NC + cid
    s_base = wid * _S_PER_W

    rows = (r0, r1, r2)
    sg = (sg0, sg1, sg2)
    ss = (ss0, ss1, ss2)

    pe_cp = pltpu.async_copy(pe_hbm.at[pl.ds(s_base, _S_PER_W)], pe_v, spe)
    for b in range(_B):
        pltpu.async_copy(ids_hbm.at[pl.ds(b * _S + s_base, _S_PER_W)],
                         idx_all.at[b], sidx)
    for b in range(_B):
        pltpu.make_async_copy(ids_hbm.at[pl.ds(0, _S_PER_W)],
                              idx_all.at[0], sidx).wait()

    def idx_ref(bt, q):
        off = pl.multiple_of(q * _CHUNK, _CHUNK)
        return idx_all.at[bt, pl.ds(off, _CHUNK)]

    pltpu.async_copy(we_hbm.at[idx_ref(0, 0)], r0, sg0)
    pltpu.async_copy(we_hbm.at[idx_ref(0, 1)], r1, sg1)
    pe_cp.wait()

    dummy_g_src = we_hbm.at[pl.ds(0, _CHUNK)]
    dummy_s_dst = out_hbm.at[pl.ds(0, _CHUNK)]

    def do_chunk(c, b):
        rb = rows[b]
        c = jnp.int32(c)
        bt = lax.shift_right_logical(c, 2)
        q = lax.bitwise_and(c, 3)
        out_row = bt * _S + s_base + q * _CHUNK
        rs = q * _CHUNK

        pltpu.make_async_copy(dummy_g_src, rb, sg[b]).wait()

        @plsc.parallel_loop(0, _VEC_PER_CHUNK, unroll=8)
        def _add(v):
            j = lax.shift_right_logical(v, 6)
            col = pl.multiple_of(lax.shift_left(lax.bitwise_and(v, 63), 4),
                                 _LANES)
            sl = pl.ds(col, _LANES)
            plsc.addupdate(rb.at[j, sl], pe_v[rs + j, sl])

        pltpu.async_copy(rb, out_hbm.at[pl.ds(out_row, _CHUNK)], ss[b])

        nb = (b + 2) % _NBUF
        nc = c + 2
        nbt = lax.shift_right_logical(nc, 2)
        nq = lax.bitwise_and(nc, 3)

        @pl.when(jnp.logical_and(nc < _NCHUNK, c >= 1))
        def _():
            pltpu.make_async_copy(rows[nb], dummy_s_dst, ss[nb]).wait()
            pltpu.async_copy(we_hbm.at[idx_ref(nbt, nq)], rows[nb], sg[nb])

        @pl.when(jnp.logical_and(nc < _NCHUNK, c < 1))
        def _():
            pltpu.async_copy(we_hbm.at[idx_ref(nbt, nq)], rows[nb], sg[nb])

    @pl.loop(0, (_NCHUNK - 1) // _NBUF)
    def _grp(gi):
        base_c = gi * _NBUF
        for b in range(_NBUF):
            do_chunk(base_c + b, b)

    do_chunk(_NCHUNK - 1, (_NCHUNK - 1) % _NBUF)

    for b in range(_NBUF):
        pltpu.make_async_copy(rows[b], dummy_s_dst, ss[b]).wait()


@jax.jit
def _emb_call(word_embeddings, ids_flat, position_embeddings):
    mesh = plsc.VectorSubcoreMesh(core_axis_name="c", subcore_axis_name="s")
    return pl.kernel(
        _emb_body,
        mesh=mesh,
        out_type=jax.ShapeDtypeStruct((_B * _S, _DIM), jnp.float32),
        scratch_types=[
            pltpu.VMEM((_B, _S_PER_W), jnp.int32),
            pltpu.VMEM((_S_PER_W, _DIM), jnp.float32),
            pltpu.VMEM((_CHUNK, _DIM), jnp.float32),
            pltpu.VMEM((_CHUNK, _DIM), jnp.float32),
            pltpu.VMEM((_CHUNK, _DIM), jnp.float32),
            pltpu.SemaphoreType.DMA,
            pltpu.SemaphoreType.DMA,
            pltpu.SemaphoreType.DMA,
            pltpu.SemaphoreType.DMA,
            pltpu.SemaphoreType.DMA,
            pltpu.SemaphoreType.DMA,
            pltpu.SemaphoreType.DMA,
            pltpu.SemaphoreType.DMA,
        ],
    )(word_embeddings, ids_flat, position_embeddings)


def kernel(input_ids, word_embeddings, position_embeddings):
    ids_flat = jnp.asarray(input_ids, jnp.int32).reshape(-1)
    out = _emb_call(word_embeddings, ids_flat, position_embeddings)
    return out.reshape(_B, _S, _DIM)
